# baseline (speedup 1.0000x reference)
import functools

import jax
import jax.numpy as jnp
from jax import lax
from jax.experimental import pallas as pl
from jax.experimental.pallas import tpu as pltpu
from jax.experimental.pallas import tpu_sc as plsc

CHUNK = 512


@functools.lru_cache(maxsize=None)
def _make_gather(n_total, d):
    info = plsc.get_sparse_core_info()
    NC, NS, L = info.num_cores, info.num_subcores, info.num_lanes
    NW = NC * NS
    assert n_total % (NW * CHUNK) == 0
    per_w = n_total // NW
    n_chunks = per_w // CHUNK
    mesh = plsc.VectorSubcoreMesh(core_axis_name="c", subcore_axis_name="s")

    @functools.partial(
        pl.kernel,
        mesh=mesh,
        compiler_params=pltpu.CompilerParams(use_tc_tiling_on_sc=False),
        out_type=jax.ShapeDtypeStruct((n_total, d), jnp.float32),
        scratch_types=[
            pltpu.VMEM((CHUNK,), jnp.int32),
            pltpu.VMEM((CHUNK, d), jnp.float32),
            pltpu.SemaphoreType.DMA,
        ],
    )
    def gather_kernel(idx_hbm, emb_hbm, out_hbm, idx_v, rows_v, sem):
        wid = lax.axis_index("s") * NC + lax.axis_index("c")
        base = wid * per_w

        def chunk_body(g, carry):
            off = pl.multiple_of(base + g * CHUNK, 8)
            pltpu.sync_copy(idx_hbm.at[pl.ds(off, CHUNK)], idx_v)

            def clamp_body(j, c):
                sl = pl.ds(j * L, L)
                idx_v[sl] = jnp.maximum(idx_v[sl], 0)
                return c

            lax.fori_loop(0, CHUNK // L, clamp_body, 0, unroll=True)
            pltpu.async_copy(emb_hbm.at[idx_v], rows_v, sem).wait()
            pltpu.sync_copy(rows_v, out_hbm.at[pl.ds(off, CHUNK)])
            return carry

        lax.fori_loop(0, n_chunks, chunk_body, 0)

    return gather_kernel


def kernel(idx, emb):
    b, l = idx.shape
    n = b * l
    d = emb.shape[1]
    flat = idx.reshape(n).astype(jnp.int32)
    out = _make_gather(n, d)(flat, emb)
    return out.reshape(b, l, d)


# speedup vs baseline: 1.0461x; 1.0461x over previous
import functools

import jax
import jax.numpy as jnp
from jax import lax
from jax.experimental import pallas as pl
from jax.experimental.pallas import tpu as pltpu
from jax.experimental.pallas import tpu_sc as plsc

CHUNK = 320
NBUF = 4


@functools.lru_cache(maxsize=None)
def _make_gather(n_total, d):
    info = plsc.get_sparse_core_info()
    NC, NS, L = info.num_cores, info.num_subcores, info.num_lanes
    NW = NC * NS
    assert n_total % (NW * CHUNK * NBUF) == 0
    per_w = n_total // NW
    n_chunks = per_w // CHUNK
    n_groups = n_chunks // NBUF
    mesh = plsc.VectorSubcoreMesh(core_axis_name="c", subcore_axis_name="s")

    @functools.partial(
        pl.kernel,
        mesh=mesh,
        compiler_params=pltpu.CompilerParams(use_tc_tiling_on_sc=False),
        out_type=jax.ShapeDtypeStruct((n_total, d), jnp.float32),
        scratch_types=(
            [pltpu.VMEM((per_w,), jnp.int32)]
            + [pltpu.VMEM((CHUNK, d), jnp.float32)] * NBUF
            + [pltpu.SemaphoreType.DMA] * (2 * NBUF)
        ),
    )
    def gather_kernel(idx_hbm, emb_hbm, out_hbm, idx_v, *bufs):
        rows = bufs[:NBUF]
        sg = bufs[NBUF : 2 * NBUF]
        ss = bufs[2 * NBUF : 3 * NBUF]
        wid = lax.axis_index("s") * NC + lax.axis_index("c")
        base = wid * per_w

        pltpu.sync_copy(idx_hbm.at[pl.ds(base, per_w)], idx_v)

        def clamp(g):
            for j in range(CHUNK // L):
                sl = pl.ds(g * CHUNK + j * L, L)
                idx_v[sl] = jnp.maximum(idx_v[sl], 0)

        def start_gather(g, b):
            pltpu.async_copy(
                emb_hbm.at[idx_v.at[pl.ds(g * CHUNK, CHUNK)]], rows[b], sg[b]
            )

        def wait_gather(g, b):
            pltpu.make_async_copy(
                emb_hbm.at[idx_v.at[pl.ds(g * CHUNK, CHUNK)]], rows[b], sg[b]
            ).wait()

        def start_store(g, b):
            pltpu.async_copy(
                rows[b], out_hbm.at[pl.ds(base + g * CHUNK, CHUNK)], ss[b]
            )

        def wait_store(g, b):
            pltpu.make_async_copy(
                rows[b], out_hbm.at[pl.ds(base + g * CHUNK, CHUNK)], ss[b]
            ).wait()

        for j in range(NBUF):
            clamp(j)
            start_gather(j, j)
            if j >= 1:
                wait_gather(j - 1, j - 1)
                start_store(j - 1, j - 1)

        def group_body(ng, carry):
            for j in range(NBUF):
                g = ng * NBUF + j
                wait_store(g - NBUF, j)
                clamp(g)
                start_gather(g, j)
                wait_gather(g - 1, (j - 1) % NBUF)
                start_store(g - 1, (j - 1) % NBUF)
            return carry

        lax.fori_loop(1, n_groups, group_body, 0)

        last = n_chunks - 1
        wait_gather(last, NBUF - 1)
        start_store(last, NBUF - 1)
        for j in range(NBUF):
            wait_store(n_chunks - NBUF + j, j)

    return gather_kernel


def kernel(idx, emb):
    b, l = idx.shape
    n = b * l
    d = emb.shape[1]
    flat = idx.reshape(n).astype(jnp.int32)
    out = _make_gather(n, d)(flat, emb)
    return out.reshape(b, l, d)


# speedup vs baseline: 1.0476x; 1.0014x over previous
import functools

import jax
import jax.numpy as jnp
from jax import lax
from jax.experimental import pallas as pl
from jax.experimental.pallas import tpu as pltpu
from jax.experimental.pallas import tpu_sc as plsc

NBUF = 8


@functools.lru_cache(maxsize=None)
def _make_gather(n_b, n_l, d):
    info = plsc.get_sparse_core_info()
    NC, NS, L = info.num_cores, info.num_subcores, info.num_lanes
    NW = NC * NS
    assert n_b % (NW * NBUF) == 0 and (n_b * n_l) % (NW * L) == 0
    b_per_w = n_b // NW
    per_w = b_per_w * n_l
    n_groups = b_per_w // NBUF
    mesh = plsc.VectorSubcoreMesh(core_axis_name="c", subcore_axis_name="s")

    @functools.partial(
        pl.kernel,
        mesh=mesh,
        compiler_params=pltpu.CompilerParams(use_tc_tiling_on_sc=False),
        out_type=jax.ShapeDtypeStruct((n_b, n_l, d), jnp.float32),
        scratch_types=(
            [pltpu.VMEM((per_w,), jnp.int32)]
            + [pltpu.VMEM((1, n_l, d), jnp.float32)] * NBUF
            + [pltpu.SemaphoreType.DMA] * (2 * NBUF)
        ),
    )
    def gather_kernel(idx_hbm, emb_hbm, out_hbm, idx_v, *bufs):
        rows = bufs[:NBUF]
        sg = bufs[NBUF : 2 * NBUF]
        ss = bufs[2 * NBUF : 3 * NBUF]
        wid = lax.axis_index("s") * NC + lax.axis_index("c")
        base = wid * per_w
        bbase = wid * b_per_w

        pltpu.sync_copy(idx_hbm.at[pl.ds(base, per_w)], idx_v)

        def clamp_all():
            def body(j, c):
                sl = pl.ds(j * L, L)
                idx_v[sl] = jnp.maximum(idx_v[sl], 0)
                return c

            lax.fori_loop(0, per_w // L, body, 0, unroll=8)

        def start_gather(g, b):
            pltpu.async_copy(
                emb_hbm.at[idx_v.at[pl.ds(g * n_l, n_l)]], rows[b].at[0], sg[b]
            )

        def wait_gather(g, b):
            pltpu.make_async_copy(
                emb_hbm.at[idx_v.at[pl.ds(g * n_l, n_l)]], rows[b].at[0], sg[b]
            ).wait()

        def start_store(g, b):
            pltpu.async_copy(rows[b], out_hbm.at[pl.ds(bbase + g, 1)], ss[b])

        def wait_store(g, b):
            pltpu.make_async_copy(
                rows[b], out_hbm.at[pl.ds(bbase + g, 1)], ss[b]
            ).wait()

        clamp_all()

        for j in range(NBUF):
            start_gather(j, j)
            if j >= 1:
                wait_gather(j - 1, j - 1)
                start_store(j - 1, j - 1)

        def group_body(ng, carry):
            for j in range(NBUF):
                g = ng * NBUF + j
                wait_store(g - NBUF, j)
                start_gather(g, j)
                wait_gather(g - 1, (j - 1) % NBUF)
                start_store(g - 1, (j - 1) % NBUF)
            return carry

        lax.fori_loop(1, n_groups, group_body, 0)

        last = b_per_w - 1
        wait_gather(last, NBUF - 1)
        start_store(last, NBUF - 1)
        for j in range(NBUF):
            wait_store(b_per_w - NBUF + j, j)

    return gather_kernel


def kernel(idx, emb):
    b, l = idx.shape
    d = emb.shape[1]
    flat = idx.reshape(b * l).astype(jnp.int32)
    return _make_gather(b, l, d)(flat, emb)
